# Initial kernel scaffold; baseline (speedup 1.0000x reference)
import jax
import jax.numpy as jnp
from jax.experimental import pallas as pl


def kernel(prediction, y):
    raise NotImplementedError("write your pallas kernel here")


# speedup vs baseline: 21.9686x; 21.9686x over previous
import functools

import jax
import jax.numpy as jnp
from jax.experimental import pallas as pl

_NEG_INF = float("-inf")
_BIG = 1 << 30


def _dlr_body(x_ref, y_ref, o_ref):
    x = x_ref[...]
    yv = y_ref[...]
    R, C = x.shape
    col = jax.lax.broadcasted_iota(jnp.int32, (R, C), 1)

    p0 = jnp.max(x, axis=1, keepdims=True)
    a0 = jnp.min(jnp.where(x == p0, col, _BIG), axis=1, keepdims=True)
    x1 = jnp.where(col == a0, _NEG_INF, x)
    p1 = jnp.max(x1, axis=1, keepdims=True)
    a1 = jnp.min(jnp.where(x1 == p1, col, _BIG), axis=1, keepdims=True)
    x2 = jnp.where(col == a1, _NEG_INF, x1)
    p2 = jnp.max(x2, axis=1, keepdims=True)

    c = jnp.max(jnp.where(col == yv, x, _NEG_INF), axis=1, keepdims=True)
    target = jnp.where(c == p0, p1, p0)
    o_ref[...] = (target - c) / (p0 - p2)


@functools.partial(jax.jit, static_argnames=("block_rows",))
def _dlr_tc(prediction, y, block_rows=256):
    n, c = prediction.shape
    y2 = y.reshape(n, 1)
    out = pl.pallas_call(
        _dlr_body,
        grid=(n // block_rows,),
        in_specs=[
            pl.BlockSpec((block_rows, c), lambda i: (i, 0)),
            pl.BlockSpec((block_rows, 1), lambda i: (i, 0)),
        ],
        out_specs=pl.BlockSpec((block_rows, 1), lambda i: (i, 0)),
        out_shape=jax.ShapeDtypeStruct((n, 1), jnp.float32),
    )(prediction, y2)
    return out.reshape(n)


def kernel(prediction, y):
    return _dlr_tc(prediction, y)
